# baseline (speedup 1.0000x reference)
import dataclasses
import functools

import numpy as np
import jax
import jax.numpy as jnp
from jax import lax
from jax.experimental import pallas as pl
from jax.experimental.pallas import tpu as pltpu
from jax.experimental.pallas import tpu_sc as plsc

N_LEVELS = 16
F = 2
LOG2_T = 19
T = 2 ** LOG2_T
BASE_RES = 16
BOUND = 1.0
PER_LEVEL_SCALE = float(np.exp2(np.log2(2048 * BOUND / 16) / (16 - 1)))
P1 = int(np.uint32(2654435761).view(np.int32))
P2 = int(np.uint32(805459861).view(np.int32))
CLIP_HI = float(np.float32(1.0 - 1e-6))

_RES = [int(np.floor(BASE_RES * (PER_LEVEL_SCALE ** l))) for l in range(N_LEVELS)]
_DENSE = [(r + 1) ** 3 <= T for r in _RES]

NW = 32
C = 1024
NIDX = 8 * C
GROWS = NIDX // 128


def _sc_encode(x0, x1, x2, tab2d, n):
    spw = n // NW
    n_chunks = spw // C
    mesh = plsc.VectorSubcoreMesh(core_axis_name="c", subcore_axis_name="s")
    cp = pltpu.CompilerParams(use_tc_tiling_on_sc=False)
    if "needs_layout_passes" in pltpu.CompilerParams.__dataclass_fields__:
        cp = dataclasses.replace(cp, needs_layout_passes=False)

    @functools.partial(
        pl.kernel, mesh=mesh, compiler_params=cp,
        out_type=jax.ShapeDtypeStruct((2 * N_LEVELS, n), jnp.float32),
        scratch_types=[
            pltpu.VMEM((C,), jnp.float32),
            pltpu.VMEM((C,), jnp.float32),
            pltpu.VMEM((C,), jnp.float32),
            pltpu.VMEM((NIDX,), jnp.int32),
            pltpu.VMEM((NIDX,), jnp.float32),
            pltpu.VMEM((NIDX, 2), jnp.float32),
            pltpu.VMEM((2 * N_LEVELS, C), jnp.float32),
            pltpu.SemaphoreType.DMA,
        ],
    )
    def enc_kernel(x0_hbm, x1_hbm, x2_hbm, tab_hbm, enc_hbm, x0_v, x1_v, x2_v,
                   idx_v, w_v, rows_v, enc_v, sem):
        wid = lax.axis_index("s") * 2 + lax.axis_index("c")
        iota16 = jnp.arange(16, dtype=jnp.int32)
        zeros16 = jnp.zeros((16,), jnp.int32)
        ones16 = jnp.ones((16,), jnp.int32)

        @pl.loop(0, n_chunks)
        def _chunk(ch):
            base = wid * spw + ch * C
            pltpu.sync_copy(x0_hbm.at[pl.ds(base, C)], x0_v)
            pltpu.sync_copy(x1_hbm.at[pl.ds(base, C)], x1_v)
            pltpu.sync_copy(x2_hbm.at[pl.ds(base, C)], x2_v)
            for l in range(N_LEVELS):
                res = _RES[l]
                dense = _DENSE[l]
                r1 = res + 1

                @pl.loop(0, C // 16)
                def _iw(j, l=l, res=res, dense=dense, r1=r1):
                    s0 = j * 16
                    p = []
                    fr = []
                    for xv_ref in (x0_v, x1_v, x2_v):
                        xv = xv_ref[pl.ds(s0, 16)]
                        xn = jnp.minimum(
                            jnp.maximum((xv + 1.0) * 0.5, 0.0),
                            jnp.float32(CLIP_HI))
                        ps = xn * jnp.float32(res)
                        pi = ps.astype(jnp.int32)
                        p.append(pi)
                        fr.append(ps - pi.astype(jnp.float32))
                    one = jnp.float32(1.0)
                    for c in range(8):
                        bx, by, bz = c & 1, (c >> 1) & 1, (c >> 2) & 1
                        cx = p[0] + 1 if bx else p[0]
                        cy = p[1] + 1 if by else p[1]
                        cz = p[2] + 1 if bz else p[2]
                        if dense:
                            idx = cx + cy * r1 + cz * (r1 * r1)
                        else:
                            idx = cx ^ (cy * jnp.int32(P1)) ^ (cz * jnp.int32(P2))
                            idx = idx & jnp.int32(T - 1)
                        idx = idx + jnp.int32(l * T)
                        wx = fr[0] if bx else one - fr[0]
                        wy = fr[1] if by else one - fr[1]
                        wz = fr[2] if bz else one - fr[2]
                        w = wx * wy * wz
                        idx_v[pl.ds(c * C + s0, 16)] = idx
                        w_v[pl.ds(c * C + s0, 16)] = w

                pltpu.async_copy(tab_hbm.at[idx_v], rows_v, sem).wait()

                @pl.loop(0, C // 16)
                def _acc(j, l=l):
                    s0 = j * 16
                    a0 = jnp.zeros((16,), jnp.float32)
                    a1 = jnp.zeros((16,), jnp.float32)
                    for c in range(8):
                        posv = (c * C) + s0 + iota16
                        w = w_v[pl.ds(c * C + s0, 16)]
                        f0 = plsc.load_gather(rows_v, [posv, zeros16])
                        f1 = plsc.load_gather(rows_v, [posv, ones16])
                        a0 = a0 + w * f0
                        a1 = a1 + w * f1
                    enc_v[2 * l, pl.ds(s0, 16)] = a0
                    enc_v[2 * l + 1, pl.ds(s0, 16)] = a1
            pltpu.sync_copy(enc_v, enc_hbm.at[:, pl.ds(base, C)])

    return enc_kernel(x0, x1, x2, tab2d)


def _sh_rows(dx, dy, dz):
    xy, xz, yz = dx * dy, dx * dz, dy * dz
    x2, y2, z2 = dx * dx, dy * dy, dz * dz
    return [
        jnp.full_like(dx, 0.28209479177387814),
        -0.48860251190291987 * dy,
        0.48860251190291992 * dz,
        -0.48860251190291987 * dx,
        1.0925484305920792 * xy,
        -1.0925484305920792 * yz,
        0.94617469575755997 * z2 - 0.31539156525251999,
        -1.0925484305920792 * xz,
        0.54627421529603959 * x2 - 0.54627421529603959 * y2,
        0.59004358992664352 * dy * (-3.0 * x2 + y2),
        2.8906114426405538 * xy * dz,
        0.45704579946446572 * dy * (1.0 - 5.0 * z2),
        0.3731763325901154 * dz * (5.0 * z2 - 3.0),
        0.45704579946446572 * dx * (1.0 - 5.0 * z2),
        1.4453057213202769 * dz * (x2 - y2),
        0.59004358992664352 * dx * (-x2 + 3.0 * y2),
    ]


def _tc_mlp(enc_t, dt, exp_c, lat_c, w0e, w0x, w0l, w1, w2, c0s, c0g, c1, c2,
            c3, n):
    NB = 2048
    grid = (n // NB,)

    def body(enc_ref, dt_ref, exp_ref, lat_ref, w0e_ref, w0x_ref, w0l_ref,
             w1_ref, w2_ref, c0s_ref, c0g_ref, c1_ref, c2_ref, c3_ref,
             sig_ref, col_ref):
        f32 = jnp.float32
        dot = functools.partial(jnp.dot, preferred_element_type=f32)
        b0 = dot(w0x_ref[...], exp_ref[...]) + dot(w0l_ref[...], lat_ref[...])
        h = dot(w0e_ref[...], enc_ref[...]) + b0
        h = jnp.maximum(h, 0.0)
        h = jnp.maximum(dot(w1_ref[...], h), 0.0)
        h2 = dot(w2_ref[...], h)
        sig_ref[...] = jnp.exp(jnp.clip(h2[0:1, :], -15.0, 15.0))
        dd0 = ((dt_ref[0:1, :] + 1.0) * 0.5) * 2.0 - 1.0
        dd1 = ((dt_ref[1:2, :] + 1.0) * 0.5) * 2.0 - 1.0
        dd2 = ((dt_ref[2:3, :] + 1.0) * 0.5) * 2.0 - 1.0
        sh = jnp.concatenate(_sh_rows(dd0, dd1, dd2), axis=0)
        g = dot(c0s_ref[...], sh) + dot(c0g_ref[...], h2)
        g = jnp.maximum(g, 0.0)
        g = jnp.maximum(dot(c1_ref[...], g), 0.0)
        g = jnp.maximum(dot(c2_ref[...], g), 0.0)
        o = dot(c3_ref[...], g)
        col_ref[...] = jax.nn.sigmoid(o)

    full = lambda shp: pl.BlockSpec(shp, lambda i: (0, 0))
    return pl.pallas_call(
        body,
        grid=grid,
        in_specs=[
            pl.BlockSpec((2 * N_LEVELS, NB), lambda i: (0, i)),
            pl.BlockSpec((3, NB), lambda i: (0, i)),
            full((79, 1)), full((32, 1)),
            full((64, 32)), full((64, 79)), full((64, 32)),
            full((64, 64)), full((16, 64)),
            full((64, 16)), full((64, 16)),
            full((64, 64)), full((64, 64)), full((3, 64)),
        ],
        out_specs=[
            pl.BlockSpec((1, NB), lambda i: (0, i)),
            pl.BlockSpec((3, NB), lambda i: (0, i)),
        ],
        out_shape=[
            jax.ShapeDtypeStruct((1, n), jnp.float32),
            jax.ShapeDtypeStruct((3, n), jnp.float32),
        ],
    )(enc_t, dt, exp_c, lat_c, w0e, w0x, w0l, w1, w2, c0s, c0g, c1, c2, c3)


def kernel(x, d, exp, latent_code, hash_table, W_s0, W_s1, W_s2, W_c0, W_c1,
           W_c2, W_c3):
    n = x.shape[0]
    xt = x.T
    dt = d.T
    tab2d = hash_table.reshape(N_LEVELS * T, F)
    enc_t = _sc_encode(xt[0], xt[1], xt[2], tab2d, n)

    exp_c = exp.reshape(-1, 1)
    lat_c = latent_code.reshape(-1, 1)
    w0e = W_s0[:32].T
    w0x = W_s0[32:32 + 79].T
    w0l = W_s0[32 + 79:].T
    w1 = W_s1.T
    w2 = W_s2.T
    c0s = W_c0[:16].T
    c0g = jnp.concatenate([jnp.zeros((64, 1), jnp.float32), W_c0[16:].T],
                          axis=1)
    c1 = W_c1.T
    c2 = W_c2.T
    c3 = W_c3.T

    sig2, colt = _tc_mlp(enc_t, dt, exp_c, lat_c, w0e, w0x, w0l, w1, w2, c0s,
                         c0g, c1, c2, c3, n)
    return (sig2.reshape(-1), colt.T)


# speedup vs baseline: 1.0228x; 1.0228x over previous
import dataclasses
import functools

import numpy as np
import jax
import jax.numpy as jnp
from jax import lax
from jax.experimental import pallas as pl
from jax.experimental.pallas import tpu as pltpu
from jax.experimental.pallas import tpu_sc as plsc

N_LEVELS = 16
F = 2
LOG2_T = 19
T = 2 ** LOG2_T
BASE_RES = 16
BOUND = 1.0
PER_LEVEL_SCALE = float(np.exp2(np.log2(2048 * BOUND / 16) / (16 - 1)))
P1 = int(np.uint32(2654435761).view(np.int32))
P2 = int(np.uint32(805459861).view(np.int32))
CLIP_HI = float(np.float32(1.0 - 1e-6))

_RES = [int(np.floor(BASE_RES * (PER_LEVEL_SCALE ** l))) for l in range(N_LEVELS)]
_DENSE = [(r + 1) ** 3 <= T for r in _RES]

NW = 32
C = 1024
NIDX = 8 * C
GROWS = NIDX // 128


def _sc_encode(x0, x1, x2, tab_flat, n):
    spw = n // NW
    n_chunks = spw // C
    mesh = plsc.VectorSubcoreMesh(core_axis_name="c", subcore_axis_name="s")
    cp = pltpu.CompilerParams(use_tc_tiling_on_sc=False)
    if "needs_layout_passes" in pltpu.CompilerParams.__dataclass_fields__:
        cp = dataclasses.replace(cp, needs_layout_passes=False)

    @functools.partial(
        pl.kernel, mesh=mesh, compiler_params=cp,
        out_type=jax.ShapeDtypeStruct((2 * N_LEVELS * n,), jnp.float32),
        scratch_types=[
            pltpu.VMEM((C,), jnp.float32),
            pltpu.VMEM((C,), jnp.float32),
            pltpu.VMEM((C,), jnp.float32),
            pltpu.VMEM((NIDX,), jnp.int32),
            pltpu.VMEM((NIDX,), jnp.int32),
            pltpu.VMEM((NIDX,), jnp.float32),
            pltpu.VMEM((NIDX,), jnp.float32),
            pltpu.VMEM((NIDX,), jnp.float32),
            pltpu.VMEM((2 * N_LEVELS, C), jnp.float32),
            pltpu.SemaphoreType.DMA,
            pltpu.SemaphoreType.DMA,
        ],
    )
    def enc_kernel(x0_hbm, x1_hbm, x2_hbm, tab_hbm, enc_hbm, x0_v, x1_v, x2_v,
                   i0_v, i1_v, w_v, r0_v, r1_v, enc_v, sem, osem):
        wid = lax.axis_index("s") * 2 + lax.axis_index("c")

        @pl.loop(0, n_chunks)
        def _chunk(ch):
            base = wid * spw + ch * C
            pltpu.sync_copy(x0_hbm.at[pl.ds(base, C)], x0_v)
            pltpu.sync_copy(x1_hbm.at[pl.ds(base, C)], x1_v)
            pltpu.sync_copy(x2_hbm.at[pl.ds(base, C)], x2_v)
            for l in range(N_LEVELS):
                res = _RES[l]
                dense = _DENSE[l]
                r1 = res + 1

                @pl.loop(0, C // 16)
                def _iw(j, l=l, res=res, dense=dense, r1=r1):
                    s0 = j * 16
                    p = []
                    fr = []
                    for xv_ref in (x0_v, x1_v, x2_v):
                        xv = xv_ref[pl.ds(s0, 16)]
                        xn = jnp.minimum(
                            jnp.maximum((xv + 1.0) * 0.5, 0.0),
                            jnp.float32(CLIP_HI))
                        ps = xn * jnp.float32(res)
                        pi = ps.astype(jnp.int32)
                        p.append(pi)
                        fr.append(ps - pi.astype(jnp.float32))
                    one = jnp.float32(1.0)
                    for c in range(8):
                        bx, by, bz = c & 1, (c >> 1) & 1, (c >> 2) & 1
                        cx = p[0] + 1 if bx else p[0]
                        cy = p[1] + 1 if by else p[1]
                        cz = p[2] + 1 if bz else p[2]
                        if dense:
                            idx = cx + cy * r1 + cz * (r1 * r1)
                        else:
                            idx = cx ^ (cy * jnp.int32(P1)) ^ (cz * jnp.int32(P2))
                            idx = idx & jnp.int32(T - 1)
                        e0 = (idx + jnp.int32(l * T)) * 2
                        wx = fr[0] if bx else one - fr[0]
                        wy = fr[1] if by else one - fr[1]
                        wz = fr[2] if bz else one - fr[2]
                        w = wx * wy * wz
                        i0_v[pl.ds(c * C + s0, 16)] = e0
                        i1_v[pl.ds(c * C + s0, 16)] = e0 + 1
                        w_v[pl.ds(c * C + s0, 16)] = w

                pltpu.async_copy(tab_hbm.at[i0_v], r0_v, sem)
                pltpu.async_copy(tab_hbm.at[i1_v], r1_v, sem)
                pltpu.make_async_copy(tab_hbm.at[i0_v], r0_v, sem).wait()
                pltpu.make_async_copy(tab_hbm.at[i1_v], r1_v, sem).wait()

                @pl.loop(0, C // 16)
                def _acc(j, l=l):
                    s0 = j * 16
                    a0 = jnp.zeros((16,), jnp.float32)
                    a1 = jnp.zeros((16,), jnp.float32)
                    for c in range(8):
                        w = w_v[pl.ds(c * C + s0, 16)]
                        a0 = a0 + w * r0_v[pl.ds(c * C + s0, 16)]
                        a1 = a1 + w * r1_v[pl.ds(c * C + s0, 16)]
                    enc_v[2 * l, pl.ds(s0, 16)] = a0
                    enc_v[2 * l + 1, pl.ds(s0, 16)] = a1

            for f in range(2 * N_LEVELS):
                pltpu.async_copy(enc_v.at[f], enc_hbm.at[pl.ds(f * n + base, C)],
                                 osem)
            for f in range(2 * N_LEVELS):
                pltpu.make_async_copy(
                    enc_v.at[f], enc_hbm.at[pl.ds(f * n + base, C)], osem).wait()

    return enc_kernel(x0, x1, x2, tab_flat)


def _sh_rows(dx, dy, dz):
    xy, xz, yz = dx * dy, dx * dz, dy * dz
    x2, y2, z2 = dx * dx, dy * dy, dz * dz
    return [
        jnp.full_like(dx, 0.28209479177387814),
        -0.48860251190291987 * dy,
        0.48860251190291992 * dz,
        -0.48860251190291987 * dx,
        1.0925484305920792 * xy,
        -1.0925484305920792 * yz,
        0.94617469575755997 * z2 - 0.31539156525251999,
        -1.0925484305920792 * xz,
        0.54627421529603959 * x2 - 0.54627421529603959 * y2,
        0.59004358992664352 * dy * (-3.0 * x2 + y2),
        2.8906114426405538 * xy * dz,
        0.45704579946446572 * dy * (1.0 - 5.0 * z2),
        0.3731763325901154 * dz * (5.0 * z2 - 3.0),
        0.45704579946446572 * dx * (1.0 - 5.0 * z2),
        1.4453057213202769 * dz * (x2 - y2),
        0.59004358992664352 * dx * (-x2 + 3.0 * y2),
    ]


def _tc_mlp(enc_t, dt, exp_c, lat_c, w0e, w0x, w0l, w1, w2, c0s, c0g, c1, c2,
            c3, n):
    NB = 2048
    grid = (n // NB,)

    def body(enc_ref, dt_ref, exp_ref, lat_ref, w0e_ref, w0x_ref, w0l_ref,
             w1_ref, w2_ref, c0s_ref, c0g_ref, c1_ref, c2_ref, c3_ref,
             sig_ref, col_ref):
        f32 = jnp.float32
        dot = functools.partial(jnp.dot, preferred_element_type=f32)
        b0 = dot(w0x_ref[...], exp_ref[...]) + dot(w0l_ref[...], lat_ref[...])
        h = dot(w0e_ref[...], enc_ref[...]) + b0
        h = jnp.maximum(h, 0.0)
        h = jnp.maximum(dot(w1_ref[...], h), 0.0)
        h2 = dot(w2_ref[...], h)
        sig_ref[...] = jnp.exp(jnp.clip(h2[0:1, :], -15.0, 15.0))
        dd0 = ((dt_ref[0:1, :] + 1.0) * 0.5) * 2.0 - 1.0
        dd1 = ((dt_ref[1:2, :] + 1.0) * 0.5) * 2.0 - 1.0
        dd2 = ((dt_ref[2:3, :] + 1.0) * 0.5) * 2.0 - 1.0
        sh = jnp.concatenate(_sh_rows(dd0, dd1, dd2), axis=0)
        g = dot(c0s_ref[...], sh) + dot(c0g_ref[...], h2)
        g = jnp.maximum(g, 0.0)
        g = jnp.maximum(dot(c1_ref[...], g), 0.0)
        g = jnp.maximum(dot(c2_ref[...], g), 0.0)
        o = dot(c3_ref[...], g)
        col_ref[...] = jax.nn.sigmoid(o)

    full = lambda shp: pl.BlockSpec(shp, lambda i: (0, 0))
    return pl.pallas_call(
        body,
        grid=grid,
        in_specs=[
            pl.BlockSpec((2 * N_LEVELS, NB), lambda i: (0, i)),
            pl.BlockSpec((3, NB), lambda i: (0, i)),
            full((79, 1)), full((32, 1)),
            full((64, 32)), full((64, 79)), full((64, 32)),
            full((64, 64)), full((16, 64)),
            full((64, 16)), full((64, 16)),
            full((64, 64)), full((64, 64)), full((3, 64)),
        ],
        out_specs=[
            pl.BlockSpec((1, NB), lambda i: (0, i)),
            pl.BlockSpec((3, NB), lambda i: (0, i)),
        ],
        out_shape=[
            jax.ShapeDtypeStruct((1, n), jnp.float32),
            jax.ShapeDtypeStruct((3, n), jnp.float32),
        ],
    )(enc_t, dt, exp_c, lat_c, w0e, w0x, w0l, w1, w2, c0s, c0g, c1, c2, c3)


def kernel(x, d, exp, latent_code, hash_table, W_s0, W_s1, W_s2, W_c0, W_c1,
           W_c2, W_c3):
    n = x.shape[0]
    xt = x.T
    dt = d.T
    tab_flat = hash_table.reshape(N_LEVELS * T * F)
    enc_flat = _sc_encode(xt[0], xt[1], xt[2], tab_flat, n)
    enc_t = enc_flat.reshape(2 * N_LEVELS, n)

    exp_c = exp.reshape(-1, 1)
    lat_c = latent_code.reshape(-1, 1)
    w0e = W_s0[:32].T
    w0x = W_s0[32:32 + 79].T
    w0l = W_s0[32 + 79:].T
    w1 = W_s1.T
    w2 = W_s2.T
    c0s = W_c0[:16].T
    c0g = jnp.concatenate([jnp.zeros((64, 1), jnp.float32), W_c0[16:].T],
                          axis=1)
    c1 = W_c1.T
    c2 = W_c2.T
    c3 = W_c3.T

    sig2, colt = _tc_mlp(enc_t, dt, exp_c, lat_c, w0e, w0x, w0l, w1, w2, c0s,
                         c0g, c1, c2, c3, n)
    return (sig2.reshape(-1), colt.T)


# speedup vs baseline: 4.0449x; 3.9547x over previous
import dataclasses
import functools

import numpy as np
import jax
import jax.numpy as jnp
from jax import lax
from jax.experimental import pallas as pl
from jax.experimental.pallas import tpu as pltpu
from jax.experimental.pallas import tpu_sc as plsc

N_LEVELS = 16
F = 2
LOG2_T = 19
T = 2 ** LOG2_T
BASE_RES = 16
BOUND = 1.0
PER_LEVEL_SCALE = float(np.exp2(np.log2(2048 * BOUND / 16) / (16 - 1)))
P1 = int(np.uint32(2654435761).view(np.int32))
P2 = int(np.uint32(805459861).view(np.int32))
CLIP_HI = float(np.float32(1.0 - 1e-6))

_RES = [int(np.floor(BASE_RES * (PER_LEVEL_SCALE ** l))) for l in range(N_LEVELS)]
_DENSE = [(r + 1) ** 3 <= T for r in _RES]

NW = 32
C = 1024
NIDX = 8 * C
GROWS = NIDX // 128


def _sc_encode(x0, x1, x2, tab_flat, n):
    spw = n // NW
    n_chunks = spw // C
    mesh = plsc.VectorSubcoreMesh(core_axis_name="c", subcore_axis_name="s")
    cp = pltpu.CompilerParams(use_tc_tiling_on_sc=False)
    if "needs_layout_passes" in pltpu.CompilerParams.__dataclass_fields__:
        cp = dataclasses.replace(cp, needs_layout_passes=False)

    @functools.partial(
        pl.kernel, mesh=mesh, compiler_params=cp,
        out_type=jax.ShapeDtypeStruct((2 * N_LEVELS * n,), jnp.float32),
        scratch_types=[
            pltpu.VMEM((C,), jnp.float32),
            pltpu.VMEM((C,), jnp.float32),
            pltpu.VMEM((C,), jnp.float32),
            pltpu.VMEM((NIDX,), jnp.int32),
            pltpu.VMEM((NIDX,), jnp.int32),
            pltpu.VMEM((NIDX,), jnp.float32),
            pltpu.VMEM((NIDX,), jnp.float32),
            pltpu.VMEM((NIDX,), jnp.float32),
            pltpu.VMEM((2 * N_LEVELS, C), jnp.float32),
            pltpu.SemaphoreType.DMA,
            pltpu.SemaphoreType.DMA,
        ],
    )
    def enc_kernel(x0_hbm, x1_hbm, x2_hbm, tab_hbm, enc_hbm, x0_v, x1_v, x2_v,
                   i0_v, i1_v, w_v, r0_v, r1_v, enc_v, sem, osem):
        wid = lax.axis_index("s") * 2 + lax.axis_index("c")

        @pl.loop(0, n_chunks)
        def _chunk(ch):
            base = wid * spw + ch * C
            pltpu.sync_copy(x0_hbm.at[pl.ds(base, C)], x0_v)
            pltpu.sync_copy(x1_hbm.at[pl.ds(base, C)], x1_v)
            pltpu.sync_copy(x2_hbm.at[pl.ds(base, C)], x2_v)
            for l in range(N_LEVELS):
                res = _RES[l]
                dense = _DENSE[l]
                r1 = res + 1

                @pl.loop(0, C // 16)
                def _iw(j, l=l, res=res, dense=dense, r1=r1):
                    s0 = j * 16
                    p = []
                    fr = []
                    for xv_ref in (x0_v, x1_v, x2_v):
                        xv = xv_ref[pl.ds(s0, 16)]
                        xn = jnp.minimum(
                            jnp.maximum((xv + 1.0) * 0.5, 0.0),
                            jnp.float32(CLIP_HI))
                        ps = xn * jnp.float32(res)
                        pi = ps.astype(jnp.int32)
                        p.append(pi)
                        fr.append(ps - pi.astype(jnp.float32))
                    one = jnp.float32(1.0)
                    for c in range(8):
                        bx, by, bz = c & 1, (c >> 1) & 1, (c >> 2) & 1
                        cx = p[0] + 1 if bx else p[0]
                        cy = p[1] + 1 if by else p[1]
                        cz = p[2] + 1 if bz else p[2]
                        if dense:
                            idx = cx + cy * r1 + cz * (r1 * r1)
                        else:
                            idx = cx ^ (cy * jnp.int32(P1)) ^ (cz * jnp.int32(P2))
                            idx = idx & jnp.int32(T - 1)
                        e0 = (jnp.int32(l * 2 * T)
                              + ((idx >> 7) << 8) + (idx & 127))
                        wx = fr[0] if bx else one - fr[0]
                        wy = fr[1] if by else one - fr[1]
                        wz = fr[2] if bz else one - fr[2]
                        w = wx * wy * wz
                        i0_v[pl.ds(c * C + s0, 16)] = e0
                        i1_v[pl.ds(c * C + s0, 16)] = e0 + 128
                        w_v[pl.ds(c * C + s0, 16)] = w

                pltpu.async_copy(tab_hbm.at[i0_v], r0_v, sem)
                pltpu.async_copy(tab_hbm.at[i1_v], r1_v, sem)
                pltpu.make_async_copy(tab_hbm.at[i0_v], r0_v, sem).wait()
                pltpu.make_async_copy(tab_hbm.at[i1_v], r1_v, sem).wait()

                @pl.loop(0, C // 16)
                def _acc(j, l=l):
                    s0 = j * 16
                    a0 = jnp.zeros((16,), jnp.float32)
                    a1 = jnp.zeros((16,), jnp.float32)
                    for c in range(8):
                        w = w_v[pl.ds(c * C + s0, 16)]
                        a0 = a0 + w * r0_v[pl.ds(c * C + s0, 16)]
                        a1 = a1 + w * r1_v[pl.ds(c * C + s0, 16)]
                    enc_v[2 * l, pl.ds(s0, 16)] = a0
                    enc_v[2 * l + 1, pl.ds(s0, 16)] = a1

            for f in range(2 * N_LEVELS):
                pltpu.async_copy(enc_v.at[f], enc_hbm.at[pl.ds(f * n + base, C)],
                                 osem)
            for f in range(2 * N_LEVELS):
                pltpu.make_async_copy(
                    enc_v.at[f], enc_hbm.at[pl.ds(f * n + base, C)], osem).wait()

    return enc_kernel(x0, x1, x2, tab_flat)


def _sh_rows(dx, dy, dz):
    xy, xz, yz = dx * dy, dx * dz, dy * dz
    x2, y2, z2 = dx * dx, dy * dy, dz * dz
    return [
        jnp.full_like(dx, 0.28209479177387814),
        -0.48860251190291987 * dy,
        0.48860251190291992 * dz,
        -0.48860251190291987 * dx,
        1.0925484305920792 * xy,
        -1.0925484305920792 * yz,
        0.94617469575755997 * z2 - 0.31539156525251999,
        -1.0925484305920792 * xz,
        0.54627421529603959 * x2 - 0.54627421529603959 * y2,
        0.59004358992664352 * dy * (-3.0 * x2 + y2),
        2.8906114426405538 * xy * dz,
        0.45704579946446572 * dy * (1.0 - 5.0 * z2),
        0.3731763325901154 * dz * (5.0 * z2 - 3.0),
        0.45704579946446572 * dx * (1.0 - 5.0 * z2),
        1.4453057213202769 * dz * (x2 - y2),
        0.59004358992664352 * dx * (-x2 + 3.0 * y2),
    ]


def _tc_mlp(enc_t, dt, exp_c, lat_c, w0e, w0x, w0l, w1, w2, c0s, c0g, c1, c2,
            c3, n):
    NB = 2048
    grid = (n // NB,)

    def body(enc_ref, dt_ref, exp_ref, lat_ref, w0e_ref, w0x_ref, w0l_ref,
             w1_ref, w2_ref, c0s_ref, c0g_ref, c1_ref, c2_ref, c3_ref,
             sig_ref, col_ref):
        f32 = jnp.float32
        dot = functools.partial(jnp.dot, preferred_element_type=f32)
        b0 = dot(w0x_ref[...], exp_ref[...]) + dot(w0l_ref[...], lat_ref[...])
        h = dot(w0e_ref[...], enc_ref[...]) + b0
        h = jnp.maximum(h, 0.0)
        h = jnp.maximum(dot(w1_ref[...], h), 0.0)
        h2 = dot(w2_ref[...], h)
        sig_ref[...] = jnp.exp(jnp.clip(h2[0:1, :], -15.0, 15.0))
        dd0 = ((dt_ref[0:1, :] + 1.0) * 0.5) * 2.0 - 1.0
        dd1 = ((dt_ref[1:2, :] + 1.0) * 0.5) * 2.0 - 1.0
        dd2 = ((dt_ref[2:3, :] + 1.0) * 0.5) * 2.0 - 1.0
        sh = jnp.concatenate(_sh_rows(dd0, dd1, dd2), axis=0)
        g = dot(c0s_ref[...], sh) + dot(c0g_ref[...], h2)
        g = jnp.maximum(g, 0.0)
        g = jnp.maximum(dot(c1_ref[...], g), 0.0)
        g = jnp.maximum(dot(c2_ref[...], g), 0.0)
        o = dot(c3_ref[...], g)
        col_ref[...] = jax.nn.sigmoid(o)

    full = lambda shp: pl.BlockSpec(shp, lambda i: (0, 0))
    return pl.pallas_call(
        body,
        grid=grid,
        in_specs=[
            pl.BlockSpec((2 * N_LEVELS, NB), lambda i: (0, i)),
            pl.BlockSpec((3, NB), lambda i: (0, i)),
            full((79, 1)), full((32, 1)),
            full((64, 32)), full((64, 79)), full((64, 32)),
            full((64, 64)), full((16, 64)),
            full((64, 16)), full((64, 16)),
            full((64, 64)), full((64, 64)), full((3, 64)),
        ],
        out_specs=[
            pl.BlockSpec((1, NB), lambda i: (0, i)),
            pl.BlockSpec((3, NB), lambda i: (0, i)),
        ],
        out_shape=[
            jax.ShapeDtypeStruct((1, n), jnp.float32),
            jax.ShapeDtypeStruct((3, n), jnp.float32),
        ],
    )(enc_t, dt, exp_c, lat_c, w0e, w0x, w0l, w1, w2, c0s, c0g, c1, c2, c3)


def kernel(x, d, exp, latent_code, hash_table, W_s0, W_s1, W_s2, W_c0, W_c1,
           W_c2, W_c3):
    n = x.shape[0]
    xt = x.T
    dt = d.T
    tab_flat = (hash_table.reshape(N_LEVELS, T // 128, 128, F)
                .transpose(0, 1, 3, 2).reshape(N_LEVELS * T * F))
    enc_flat = _sc_encode(xt[0], xt[1], xt[2], tab_flat, n)
    enc_t = enc_flat.reshape(2 * N_LEVELS, n)

    exp_c = exp.reshape(-1, 1)
    lat_c = latent_code.reshape(-1, 1)
    w0e = W_s0[:32].T
    w0x = W_s0[32:32 + 79].T
    w0l = W_s0[32 + 79:].T
    w1 = W_s1.T
    w2 = W_s2.T
    c0s = W_c0[:16].T
    c0g = jnp.concatenate([jnp.zeros((64, 1), jnp.float32), W_c0[16:].T],
                          axis=1)
    c1 = W_c1.T
    c2 = W_c2.T
    c3 = W_c3.T

    sig2, colt = _tc_mlp(enc_t, dt, exp_c, lat_c, w0e, w0x, w0l, w1, w2, c0s,
                         c0g, c1, c2, c3, n)
    return (sig2.reshape(-1), colt.T)
